# baseline (speedup 1.0000x reference)
import functools

import jax
import jax.numpy as jnp
from jax import lax
from jax.experimental import pallas as pl
from jax.experimental.pallas import tpu as pltpu
from jax.experimental.pallas import tpu_sc as plsc

NC = 2
NS = 16
L = 16
D = 512
B = 16
SPC = B // NC
BLK = 64
H = D // 2
CH = H // L


def kernel(stack_lengths, features):
    mesh = plsc.VectorSubcoreMesh(core_axis_name="c", subcore_axis_name="s")

    @functools.partial(
        pl.kernel,
        out_type=jax.ShapeDtypeStruct((B, D), jnp.float32),
        mesh=mesh,
        scratch_types=[
            pltpu.VMEM((1, L), jnp.int32),
            pltpu.VMEM((2, BLK, D), jnp.float32),
            pltpu.VMEM((B + 1, D), jnp.float32),
            pltpu.VMEM((NS, D), jnp.float32),
            pltpu.VMEM_SHARED((SPC, NS, D), jnp.float32),
            pltpu.SemaphoreType.DMA,
            pltpu.SemaphoreType.DMA,
        ],
        compiler_params=pltpu.CompilerParams(needs_layout_passes=False),
    )
    def k(lens_hbm, feat_hbm, out_hbm, lens_v, bufs, acc, tmp, shared,
          sem0, sem1):
        cid = lax.axis_index("c")
        sid = lax.axis_index("s")

        pltpu.sync_copy(lens_hbm, lens_v)
        lens = lens_v[0, :]
        ends = jnp.cumsum(lens)
        iota = lax.iota(jnp.int32, L)
        e = [jnp.max(jnp.where(iota == j, ends, 0)) for j in range(B)]

        slo = cid * SPC
        shi = slo + SPC
        row_lo = jnp.where(cid == 0, 0, e[SPC - 1])
        row_hi = jnp.where(cid == 0, e[SPC - 1], e[B - 1])

        fb = row_lo // BLK
        lb = (row_hi + BLK - 1) // BLK
        nmine = (lb - fb - sid + NS - 1) // NS
        sems = (sem0, sem1)

        def blk_base(i):
            return (fb + sid + i * NS) * BLK

        for b in range(2):
            @pl.when(nmine > b)
            def _(b=b):
                pltpu.async_copy(
                    feat_hbm.at[pl.ds(blk_base(b), BLK)], bufs.at[b], sems[b]
                )

        zero = jnp.zeros((L,), jnp.float32)
        for r in range(B + 1):
            for c in range(D // L):
                acc[r, pl.ds(c * L, L)] = zero

        def consume(i, buf):
            base = blk_base(i)

            def run_cond(st):
                return st[0] < BLK

            def run_body(st):
                r = st[0]
                row = base + r
                s = jnp.int32(0)
                re = jnp.int32(BLK)
                for j in range(B):
                    s = s + jnp.where(e[j] <= row, 1, 0)
                    ej_rel = e[j] - base
                    re = jnp.where(
                        jnp.logical_and(e[j] > row, ej_rel < re), ej_rel, re
                    )
                s = jnp.where(
                    jnp.logical_and(s >= slo, s < shi), s, jnp.int32(B)
                )
                for h in range(2):
                    col0 = h * H

                    def inner(rr, vs):
                        return tuple(
                            vs[c] + buf[rr, pl.ds(col0 + c * L, L)]
                            for c in range(CH)
                        )

                    init = tuple(
                        jnp.zeros((L,), jnp.float32) for _ in range(CH)
                    )
                    vs = lax.fori_loop(r, re, inner, init)
                    for c in range(CH):
                        plsc.addupdate(
                            acc.at[s, pl.ds(col0 + c * L, L)], vs[c]
                        )
                return (re,)

            lax.while_loop(run_cond, run_body, (jnp.int32(0),))

        def pair_body(p, carry):
            for b in range(2):
                i = 2 * p + b

                @pl.when(i < nmine)
                def _(i=i, b=b):
                    pltpu.make_async_copy(
                        feat_hbm.at[pl.ds(0, BLK)], bufs.at[b], sems[b]
                    ).wait()
                    consume(i, bufs.at[b])

                    @pl.when(i + 2 < nmine)
                    def _():
                        pltpu.async_copy(
                            feat_hbm.at[pl.ds(blk_base(i + 2), BLK)],
                            bufs.at[b],
                            sems[b],
                        )
            return carry

        lax.fori_loop(0, (nmine + 1) // 2, pair_body, 0)

        for ls in range(SPC):
            pltpu.sync_copy(acc.at[slo + ls], shared.at[ls, sid])
        plsc.subcore_barrier()

        @pl.when(sid < SPC)
        def _():
            g = slo + sid
            pltpu.sync_copy(shared.at[sid], tmp)
            ln = jnp.max(jnp.where(iota == g, lens, 0)).astype(jnp.float32)
            for c in range(D // L):
                v = tmp[0, pl.ds(c * L, L)]
                for w in range(1, NS):
                    v = v + tmp[w, pl.ds(c * L, L)]
                tmp[0, pl.ds(c * L, L)] = v / ln
            pltpu.sync_copy(tmp.at[0], out_hbm.at[g])

    return k(stack_lengths, features)


# speedup vs baseline: 1.0110x; 1.0110x over previous
import functools

import jax
import jax.numpy as jnp
from jax import lax
from jax.experimental import pallas as pl
from jax.experimental.pallas import tpu as pltpu
from jax.experimental.pallas import tpu_sc as plsc

NC = 2
NS = 16
L = 16
D = 512
B = 16
SPC = B // NC
BLK = 64
H = D // 2
CH = H // L


def kernel(stack_lengths, features):
    mesh = plsc.VectorSubcoreMesh(core_axis_name="c", subcore_axis_name="s")

    @functools.partial(
        pl.kernel,
        out_type=jax.ShapeDtypeStruct((B, D), jnp.float32),
        mesh=mesh,
        scratch_types=[
            pltpu.VMEM((1, L), jnp.int32),
            pltpu.VMEM((2, BLK, D), jnp.float32),
            pltpu.VMEM((B + 1, D), jnp.float32),
            pltpu.VMEM((NS, D), jnp.float32),
            pltpu.VMEM_SHARED((B, NS, D), jnp.float32),
            pltpu.SemaphoreType.DMA,
            pltpu.SemaphoreType.DMA,
        ],
        compiler_params=pltpu.CompilerParams(needs_layout_passes=False),
    )
    def k(lens_hbm, feat_hbm, out_hbm, lens_v, bufs, acc, tmp, shared,
          sem0, sem1):
        cid = lax.axis_index("c")
        sid = lax.axis_index("s")

        pltpu.sync_copy(lens_hbm, lens_v)
        lens = lens_v[0, :]
        ends = jnp.cumsum(lens)
        iota = lax.iota(jnp.int32, L)
        e = [jnp.max(jnp.where(iota == j, ends, 0)) for j in range(B)]

        total = e[B - 1]
        half = total // 2
        m = jnp.int32(0)
        for j in range(B):
            m = m + jnp.where(e[j] < half, 1, 0)
        e_m1 = jnp.max(jnp.where(iota == m - 1, ends, 0))
        e_m = jnp.max(jnp.where(iota == m, ends, 0))
        m = m + jnp.where(half - e_m1 > e_m - half, 1, 0)
        slo = jnp.where(cid == 0, 0, m)
        shi = jnp.where(cid == 0, m, B)
        row_lo = jnp.where(cid == 0, 0, jnp.max(jnp.where(iota == m - 1, ends, 0)))
        row_hi = jnp.where(cid == 0, jnp.max(jnp.where(iota == m - 1, ends, 0)), total)

        fb = row_lo // BLK
        lb = (row_hi + BLK - 1) // BLK
        nmine = (lb - fb - sid + NS - 1) // NS
        sems = (sem0, sem1)

        def blk_base(i):
            return (fb + sid + i * NS) * BLK

        for b in range(2):
            @pl.when(nmine > b)
            def _(b=b):
                pltpu.async_copy(
                    feat_hbm.at[pl.ds(blk_base(b), BLK)], bufs.at[b], sems[b]
                )

        zero = jnp.zeros((L,), jnp.float32)
        for r in range(B + 1):
            for c in range(D // L):
                acc[r, pl.ds(c * L, L)] = zero

        def consume(i, buf):
            base = blk_base(i)

            def run_cond(st):
                return st[0] < BLK

            def run_body(st):
                r = st[0]
                row = base + r
                s = jnp.int32(0)
                re = jnp.int32(BLK)
                for j in range(B):
                    s = s + jnp.where(e[j] <= row, 1, 0)
                    ej_rel = e[j] - base
                    re = jnp.where(
                        jnp.logical_and(e[j] > row, ej_rel < re), ej_rel, re
                    )
                s = jnp.where(
                    jnp.logical_and(s >= slo, s < shi), s, jnp.int32(B)
                )
                for h in range(2):
                    col0 = h * H

                    def inner(rr, vs):
                        return tuple(
                            vs[c] + buf[rr, pl.ds(col0 + c * L, L)]
                            for c in range(CH)
                        )

                    init = tuple(
                        jnp.zeros((L,), jnp.float32) for _ in range(CH)
                    )
                    vs = lax.fori_loop(r, re, inner, init)
                    for c in range(CH):
                        plsc.addupdate(
                            acc.at[s, pl.ds(col0 + c * L, L)], vs[c]
                        )
                return (re,)

            lax.while_loop(run_cond, run_body, (jnp.int32(0),))

        def pair_body(p, carry):
            for b in range(2):
                i = 2 * p + b

                @pl.when(i < nmine)
                def _(i=i, b=b):
                    pltpu.make_async_copy(
                        feat_hbm.at[pl.ds(0, BLK)], bufs.at[b], sems[b]
                    ).wait()
                    consume(i, bufs.at[b])

                    @pl.when(i + 2 < nmine)
                    def _():
                        pltpu.async_copy(
                            feat_hbm.at[pl.ds(blk_base(i + 2), BLK)],
                            bufs.at[b],
                            sems[b],
                        )
            return carry

        lax.fori_loop(0, (nmine + 1) // 2, pair_body, 0)

        for ls in range(B):
            pltpu.sync_copy(acc.at[ls], shared.at[ls, sid])
        plsc.subcore_barrier()

        @pl.when(jnp.logical_and(sid >= slo, sid < shi))
        def _():
            pltpu.sync_copy(shared.at[sid], tmp)
            ln = jnp.max(jnp.where(iota == sid, lens, 0)).astype(jnp.float32)
            for c in range(D // L):
                v = tmp[0, pl.ds(c * L, L)]
                for w in range(1, NS):
                    v = v + tmp[w, pl.ds(c * L, L)]
                tmp[0, pl.ds(c * L, L)] = v / ln
            pltpu.sync_copy(tmp.at[0], out_hbm.at[sid])

    return k(stack_lengths, features)
